# baseline (speedup 1.0000x reference)
import functools

import jax
import jax.numpy as jnp
from jax import lax
from jax.experimental import pallas as pl
from jax.experimental.pallas import tpu as pltpu
from jax.experimental.pallas import tpu_sc as plsc

B = 1024
EMB = 64
VOCAB = 100000

VPAD = 100352
RB = 16
NEG = -1e30


def _sc_gather(idx2, table2):
    info = plsc.get_sparse_core_info()
    nw = info.num_cores * info.num_subcores
    bpw = B // nw
    mesh = plsc.VectorSubcoreMesh(core_axis_name="c", subcore_axis_name="s")

    @functools.partial(
        pl.kernel,
        mesh=mesh,
        out_type=jax.ShapeDtypeStruct((B, 2 * EMB), jnp.float32),
        scratch_types=[
            pltpu.VMEM((bpw,), jnp.int32),
            pltpu.VMEM((bpw, 2 * EMB), jnp.float32),
            pltpu.SemaphoreType.DMA,
        ],
    )
    def gather_kernel(idx_hbm, table_hbm, out_hbm, idx_v, rows_v, sem):
        wid = lax.axis_index("s") * info.num_cores + lax.axis_index("c")
        base = wid * bpw
        pltpu.sync_copy(idx_hbm.at[pl.ds(base, bpw)], idx_v)
        pltpu.async_copy(table_hbm.at[idx_v], rows_v, sem).wait()
        pltpu.sync_copy(rows_v, out_hbm.at[pl.ds(base, bpw)])

    return gather_kernel(idx2, table2)


def _tc_body(h2_ref, par_ref, w_ref, b_ref, out_ref):
    hsel = jnp.where(par_ref[...] == 0,
                     h2_ref[:, :EMB], h2_ref[:, EMB:])
    hs = jnp.maximum(hsel, 0.0).astype(jnp.bfloat16)
    logits = lax.dot_general(
        hs, w_ref[...], (((1,), (1,)), ((), ())),
        preferred_element_type=jnp.float32,
    ) + b_ref[...]
    m = jnp.max(logits, axis=1, keepdims=True)
    s = jnp.sum(jnp.exp(logits - m), axis=1, keepdims=True)
    out_ref[...] = logits - (m + jnp.log(s))


def _tc_logsoftmax(h2, par, Wp, bp):
    return pl.pallas_call(
        _tc_body,
        grid=(B // RB,),
        in_specs=[
            pl.BlockSpec((RB, 2 * EMB), lambda i: (i, 0)),
            pl.BlockSpec((RB, 1), lambda i: (i, 0)),
            pl.BlockSpec((VPAD, EMB), lambda i: (0, 0)),
            pl.BlockSpec((1, VPAD), lambda i: (0, 0)),
        ],
        out_specs=pl.BlockSpec((RB, VPAD), lambda i: (i, 0)),
        out_shape=jax.ShapeDtypeStruct((B, VOCAB), jnp.float32),
    )(h2, par, Wp, bp)


def kernel(input, table, W, b):
    idx = input.astype(jnp.int32)
    table2 = table.reshape(VOCAB // 2, 2 * EMB)
    h2 = _sc_gather(idx >> 1, table2)
    par = (idx & 1).astype(jnp.float32).reshape(B, 1)
    Wp = jnp.pad(W.astype(jnp.bfloat16), ((0, VPAD - VOCAB), (0, 0)))
    bp = jnp.pad(b, (0, VPAD - VOCAB),
                 constant_values=NEG).reshape(1, VPAD)
    return _tc_logsoftmax(h2, par, Wp, bp)


# speedup vs baseline: 1.2567x; 1.2567x over previous
import functools

import jax
import jax.numpy as jnp
from jax import lax
from jax.experimental import pallas as pl
from jax.experimental.pallas import tpu as pltpu
from jax.experimental.pallas import tpu_sc as plsc

B = 1024
EMB = 64
VOCAB = 100000

VT = 2048
NT = (VOCAB + VT - 1) // VT
VPAD = NT * VT
RB = 32
NEG = -1e30


def _sc_gather(idx2, table2):
    info = plsc.get_sparse_core_info()
    nw = info.num_cores * info.num_subcores
    bpw = B // nw
    mesh = plsc.VectorSubcoreMesh(core_axis_name="c", subcore_axis_name="s")

    @functools.partial(
        pl.kernel,
        mesh=mesh,
        out_type=jax.ShapeDtypeStruct((B, 2 * EMB), jnp.float32),
        scratch_types=[
            pltpu.VMEM((bpw,), jnp.int32),
            pltpu.VMEM((bpw, 2 * EMB), jnp.float32),
            pltpu.SemaphoreType.DMA,
        ],
    )
    def gather_kernel(idx_hbm, table_hbm, out_hbm, idx_v, rows_v, sem):
        wid = lax.axis_index("s") * info.num_cores + lax.axis_index("c")
        base = wid * bpw
        pltpu.sync_copy(idx_hbm.at[pl.ds(base, bpw)], idx_v)
        pltpu.async_copy(table_hbm.at[idx_v], rows_v, sem).wait()
        pltpu.sync_copy(rows_v, out_hbm.at[pl.ds(base, bpw)])

    return gather_kernel(idx2, table2)


def _pass_a(h2_ref, par_ref, w_ref, b_ref, lse_ref, slab_ref,
            hs_ref, m_ref, s_ref):
    j = pl.program_id(0)

    @pl.when(j == 0)
    def _prep():
        hsel = jnp.where(par_ref[...] == 0,
                         h2_ref[:, :EMB], h2_ref[:, EMB:])
        hs_ref[...] = jnp.maximum(hsel, 0.0).astype(jnp.bfloat16)
        m_ref[...] = jnp.full_like(m_ref, NEG)
        s_ref[...] = jnp.zeros_like(s_ref)

    logits = lax.dot_general(
        hs_ref[...], w_ref[...], (((1,), (1,)), ((), ())),
        preferred_element_type=jnp.float32,
    ) + b_ref[...]
    slab_ref[0] = logits.astype(jnp.bfloat16)

    m_old = m_ref[...]
    m_new = jnp.maximum(m_old, jnp.max(logits, axis=1, keepdims=True))
    s_ref[...] = (s_ref[...] * jnp.exp(m_old - m_new)
                  + jnp.sum(jnp.exp(logits - m_new), axis=1, keepdims=True))
    m_ref[...] = m_new

    @pl.when(j == pl.num_programs(0) - 1)
    def _finalize():
        lse_ref[...] = m_ref[...] + jnp.log(s_ref[...])


def _pass_b(slab_ref, lse_ref, out_ref):
    lse = lse_ref[...]
    for j in range(NT):
        out_ref[:, j * VT:(j + 1) * VT] = (
            slab_ref[j].astype(jnp.float32) - lse)


def _tc_logsoftmax(h2, par, Wp, bp):
    lse, slab = pl.pallas_call(
        _pass_a,
        grid=(NT,),
        in_specs=[
            pl.BlockSpec((B, 2 * EMB), lambda j: (0, 0)),
            pl.BlockSpec((B, 1), lambda j: (0, 0)),
            pl.BlockSpec((VT, EMB), lambda j: (j, 0)),
            pl.BlockSpec((1, VT), lambda j: (0, j)),
        ],
        out_specs=[
            pl.BlockSpec((B, 1), lambda j: (0, 0)),
            pl.BlockSpec((1, B, VT), lambda j: (j, 0, 0)),
        ],
        out_shape=[
            jax.ShapeDtypeStruct((B, 1), jnp.float32),
            jax.ShapeDtypeStruct((NT, B, VT), jnp.bfloat16),
        ],
        scratch_shapes=[
            pltpu.VMEM((B, EMB), jnp.bfloat16),
            pltpu.VMEM((B, 1), jnp.float32),
            pltpu.VMEM((B, 1), jnp.float32),
        ],
    )(h2, par, Wp, bp)

    return pl.pallas_call(
        _pass_b,
        grid=(B // RB,),
        in_specs=[
            pl.BlockSpec((NT, RB, VT), lambda s: (0, s, 0)),
            pl.BlockSpec((RB, 1), lambda s: (s, 0)),
        ],
        out_specs=pl.BlockSpec((RB, VPAD), lambda s: (s, 0)),
        out_shape=jax.ShapeDtypeStruct((B, VOCAB), jnp.float32),
    )(slab, lse)


def kernel(input, table, W, b):
    idx = input.astype(jnp.int32)
    table2 = table.reshape(VOCAB // 2, 2 * EMB)
    h2 = _sc_gather(idx >> 1, table2)
    par = (idx & 1).astype(jnp.float32).reshape(B, 1)
    Wp = jnp.pad(W.astype(jnp.bfloat16), ((0, VPAD - VOCAB), (0, 0)))
    bp = jnp.pad(b, (0, VPAD - VOCAB),
                 constant_values=NEG).reshape(1, VPAD)
    return _tc_logsoftmax(h2, par, Wp, bp)


# speedup vs baseline: 2.3660x; 1.8827x over previous
import functools

import jax
import jax.numpy as jnp
from jax import lax
from jax.experimental import pallas as pl
from jax.experimental.pallas import tpu as pltpu
from jax.experimental.pallas import tpu_sc as plsc

B = 1024
EMB = 64
VOCAB = 100000

VT = 2048
NT = (VOCAB + VT - 1) // VT
VPAD = NT * VT
NEG = -1e30


def _sc_gather(idx2, table2):
    info = plsc.get_sparse_core_info()
    nw = info.num_cores * info.num_subcores
    bpw = B // nw
    mesh = plsc.VectorSubcoreMesh(core_axis_name="c", subcore_axis_name="s")

    @functools.partial(
        pl.kernel,
        mesh=mesh,
        out_type=jax.ShapeDtypeStruct((B, 2 * EMB), jnp.float32),
        scratch_types=[
            pltpu.VMEM((bpw,), jnp.int32),
            pltpu.VMEM((bpw, 2 * EMB), jnp.float32),
            pltpu.SemaphoreType.DMA,
        ],
    )
    def gather_kernel(idx_hbm, table_hbm, out_hbm, idx_v, rows_v, sem):
        wid = lax.axis_index("s") * info.num_cores + lax.axis_index("c")
        base = wid * bpw
        pltpu.sync_copy(idx_hbm.at[pl.ds(base, bpw)], idx_v)
        pltpu.async_copy(table_hbm.at[idx_v], rows_v, sem).wait()
        pltpu.sync_copy(rows_v, out_hbm.at[pl.ds(base, bpw)])

    return gather_kernel(idx2, table2)


def _logits_t(hs_ref, w_ref, b_ref):
    return lax.dot_general(
        w_ref[...], hs_ref[...], (((1,), (1,)), ((), ())),
        preferred_element_type=jnp.float32,
    ) + b_ref[...]


def _stats_body(h2_ref, par_ref, w_ref, b_ref, lse_ref, hs_out_ref,
                hs_ref, m_ref, s_ref):
    j = pl.program_id(0)

    @pl.when(j == 0)
    def _prep():
        hsel = jnp.where(par_ref[...] == 0,
                         h2_ref[:, :EMB], h2_ref[:, EMB:])
        hs = jnp.maximum(hsel, 0.0).astype(jnp.bfloat16)
        hs_ref[...] = hs
        hs_out_ref[...] = hs
        m_ref[...] = jnp.full_like(m_ref, NEG)
        s_ref[...] = jnp.zeros_like(s_ref)

    logits = _logits_t(hs_ref, w_ref, b_ref)

    m_old = m_ref[...]
    m_new = jnp.maximum(m_old, jnp.max(logits, axis=0, keepdims=True))
    s_ref[...] = (s_ref[...] * jnp.exp(m_old - m_new)
                  + jnp.sum(jnp.exp(logits - m_new), axis=0, keepdims=True))
    m_ref[...] = m_new

    @pl.when(j == pl.num_programs(0) - 1)
    def _finalize():
        lse_ref[...] = m_ref[...] + jnp.log(s_ref[...])


def _write_body(hs_ref, w_ref, b_ref, lse_ref, out_ref):
    out_ref[...] = _logits_t(hs_ref, w_ref, b_ref) - lse_ref[...]


def _tc_logsoftmax(h2, par, Wp, bp):
    lse, hs = pl.pallas_call(
        _stats_body,
        grid=(NT,),
        in_specs=[
            pl.BlockSpec((B, 2 * EMB), lambda j: (0, 0)),
            pl.BlockSpec((B, 1), lambda j: (0, 0)),
            pl.BlockSpec((VT, EMB), lambda j: (j, 0)),
            pl.BlockSpec((VT, 1), lambda j: (j, 0)),
        ],
        out_specs=[
            pl.BlockSpec((1, B), lambda j: (0, 0)),
            pl.BlockSpec((B, EMB), lambda j: (0, 0)),
        ],
        out_shape=[
            jax.ShapeDtypeStruct((1, B), jnp.float32),
            jax.ShapeDtypeStruct((B, EMB), jnp.bfloat16),
        ],
        scratch_shapes=[
            pltpu.VMEM((B, EMB), jnp.bfloat16),
            pltpu.VMEM((1, B), jnp.float32),
            pltpu.VMEM((1, B), jnp.float32),
        ],
    )(h2, par, Wp, bp)

    out_t = pl.pallas_call(
        _write_body,
        grid=(NT,),
        in_specs=[
            pl.BlockSpec((B, EMB), lambda j: (0, 0)),
            pl.BlockSpec((VT, EMB), lambda j: (j, 0)),
            pl.BlockSpec((VT, 1), lambda j: (j, 0)),
            pl.BlockSpec((1, B), lambda j: (0, 0)),
        ],
        out_specs=pl.BlockSpec((VT, B), lambda j: (j, 0)),
        out_shape=jax.ShapeDtypeStruct((VOCAB, B), jnp.float32),
    )(hs, Wp, bp, lse)
    return out_t.T


def kernel(input, table, W, b):
    idx = input.astype(jnp.int32)
    table2 = table.reshape(VOCAB // 2, 2 * EMB)
    h2 = _sc_gather(idx >> 1, table2)
    par = (idx & 1).astype(jnp.float32).reshape(B, 1)
    Wp = jnp.pad(W.astype(jnp.bfloat16), ((0, VPAD - VOCAB), (0, 0)))
    bp = jnp.pad(b, (0, VPAD - VOCAB),
                 constant_values=NEG).reshape(VPAD, 1)
    return _tc_logsoftmax(h2, par, Wp, bp)


# speedup vs baseline: 2.3995x; 1.0141x over previous
import functools

import jax
import jax.numpy as jnp
from jax import lax
from jax.experimental import pallas as pl
from jax.experimental.pallas import tpu as pltpu
from jax.experimental.pallas import tpu_sc as plsc

B = 1024
EMB = 64
VOCAB = 100000

VT = 4096
NT = (VOCAB + VT - 1) // VT
VPAD = NT * VT
NEG = -1e30


def _sc_gather(idx2, table2):
    info = plsc.get_sparse_core_info()
    nw = info.num_cores * info.num_subcores
    bpw = B // nw
    mesh = plsc.VectorSubcoreMesh(core_axis_name="c", subcore_axis_name="s")

    @functools.partial(
        pl.kernel,
        mesh=mesh,
        out_type=jax.ShapeDtypeStruct((B, 2 * EMB), jnp.float32),
        scratch_types=[
            pltpu.VMEM((bpw,), jnp.int32),
            pltpu.VMEM((bpw, 2 * EMB), jnp.float32),
            pltpu.SemaphoreType.DMA,
        ],
    )
    def gather_kernel(idx_hbm, table_hbm, out_hbm, idx_v, rows_v, sem):
        wid = lax.axis_index("s") * info.num_cores + lax.axis_index("c")
        base = wid * bpw
        pltpu.sync_copy(idx_hbm.at[pl.ds(base, bpw)], idx_v)
        pltpu.async_copy(table_hbm.at[idx_v], rows_v, sem).wait()
        pltpu.sync_copy(rows_v, out_hbm.at[pl.ds(base, bpw)])

    return gather_kernel(idx2, table2)


def _logits_t(hs_ref, w_ref, b_ref):
    return lax.dot_general(
        w_ref[...], hs_ref[...], (((1,), (1,)), ((), ())),
        preferred_element_type=jnp.float32,
    ) + b_ref[...]


def _stats_body(h2_ref, par_ref, w_ref, b_ref, lse_ref, hs_out_ref,
                hs_ref, m_ref, s_ref):
    j = pl.program_id(0)

    @pl.when(j == 0)
    def _prep():
        hsel = jnp.where(par_ref[...] == 0,
                         h2_ref[:, :EMB], h2_ref[:, EMB:])
        hs = jnp.maximum(hsel, 0.0).astype(jnp.bfloat16)
        hs_ref[...] = hs
        hs_out_ref[...] = hs
        m_ref[...] = jnp.full_like(m_ref, NEG)
        s_ref[...] = jnp.zeros_like(s_ref)

    logits = _logits_t(hs_ref, w_ref, b_ref)

    m_old = m_ref[...]
    m_new = jnp.maximum(m_old, jnp.max(logits, axis=0, keepdims=True))
    s_ref[...] = (s_ref[...] * jnp.exp(m_old - m_new)
                  + jnp.sum(jnp.exp(logits - m_new), axis=0, keepdims=True))
    m_ref[...] = m_new

    @pl.when(j == pl.num_programs(0) - 1)
    def _finalize():
        lse_ref[...] = m_ref[...] + jnp.log(s_ref[...])


def _write_body(hs_ref, w_ref, b_ref, lse_ref, out_ref):
    out_ref[...] = _logits_t(hs_ref, w_ref, b_ref) - lse_ref[...]


def _tc_logsoftmax(h2, par, Wp, bp):
    lse, hs = pl.pallas_call(
        _stats_body,
        grid=(NT,),
        in_specs=[
            pl.BlockSpec((B, 2 * EMB), lambda j: (0, 0)),
            pl.BlockSpec((B, 1), lambda j: (0, 0)),
            pl.BlockSpec((VT, EMB), lambda j: (j, 0)),
            pl.BlockSpec((VT, 1), lambda j: (j, 0)),
        ],
        out_specs=[
            pl.BlockSpec((1, B), lambda j: (0, 0)),
            pl.BlockSpec((B, EMB), lambda j: (0, 0)),
        ],
        out_shape=[
            jax.ShapeDtypeStruct((1, B), jnp.float32),
            jax.ShapeDtypeStruct((B, EMB), jnp.bfloat16),
        ],
        scratch_shapes=[
            pltpu.VMEM((B, EMB), jnp.bfloat16),
            pltpu.VMEM((1, B), jnp.float32),
            pltpu.VMEM((1, B), jnp.float32),
        ],
    )(h2, par, Wp, bp)

    out_t = pl.pallas_call(
        _write_body,
        grid=(NT,),
        in_specs=[
            pl.BlockSpec((B, EMB), lambda j: (0, 0)),
            pl.BlockSpec((VT, EMB), lambda j: (j, 0)),
            pl.BlockSpec((VT, 1), lambda j: (j, 0)),
            pl.BlockSpec((1, B), lambda j: (0, 0)),
        ],
        out_specs=pl.BlockSpec((VT, B), lambda j: (j, 0)),
        out_shape=jax.ShapeDtypeStruct((VOCAB, B), jnp.float32),
    )(hs, Wp, bp, lse)
    return out_t.T


def kernel(input, table, W, b):
    idx = input.astype(jnp.int32)
    table2 = table.reshape(VOCAB // 2, 2 * EMB)
    h2 = _sc_gather(idx >> 1, table2)
    par = (idx & 1).astype(jnp.float32).reshape(B, 1)
    Wp = jnp.pad(W.astype(jnp.bfloat16), ((0, VPAD - VOCAB), (0, 0)))
    bp = jnp.pad(b, (0, VPAD - VOCAB),
                 constant_values=NEG).reshape(VPAD, 1)
    return _tc_logsoftmax(h2, par, Wp, bp)
